# baseline (speedup 1.0000x reference)
import functools

import jax
import jax.numpy as jnp
from jax import lax
from jax.experimental import pallas as pl
from jax.experimental.pallas import tpu as pltpu
from jax.experimental.pallas import tpu_sc as plsc

BATCH = 16384
NUM_FIELDS = 26
TOTAL_ROWS = 100000 * 26

NUM_CORES = 2
NUM_SUBCORES = 16
LANES = 16
NUM_WORKERS = NUM_CORES * NUM_SUBCORES

B_PER_W = BATCH // NUM_WORKERS
IDX_PER_W = B_PER_W * NUM_FIELDS
B_CHUNKS = B_PER_W // LANES
CHUNK_IDX = NUM_FIELDS * LANES
NUM_SEGS = 8
SEG_CHUNKS = B_CHUNKS // NUM_SEGS
SEG_IDX = SEG_CHUNKS * CHUNK_IDX


@functools.partial(
    pl.kernel,
    out_type=jax.ShapeDtypeStruct((BATCH,), jnp.float32),
    mesh=plsc.VectorSubcoreMesh(core_axis_name="c", subcore_axis_name="s"),
    compiler_params=pltpu.CompilerParams(needs_layout_passes=False),
    scratch_types=[
        pltpu.VMEM((NUM_FIELDS, B_PER_W), jnp.int32),
        pltpu.VMEM((IDX_PER_W,), jnp.int32),
        pltpu.VMEM((IDX_PER_W,), jnp.float32),
        pltpu.VMEM((8 + NUM_FIELDS,), jnp.int32),
        pltpu.VMEM((16,), jnp.float32),
        pltpu.VMEM((B_PER_W,), jnp.float32),
        pltpu.SemaphoreType.DMA,
        pltpu.SemaphoreType.DMA,
        pltpu.SemaphoreType.DMA,
        pltpu.SemaphoreType.DMA,
        pltpu.SemaphoreType.DMA,
        pltpu.SemaphoreType.DMA,
        pltpu.SemaphoreType.DMA,
        pltpu.SemaphoreType.DMA,
        pltpu.SemaphoreType.DMA,
    ],
)
def _fl_kernel(xt_hbm, offs_hbm, table_hbm, bias_hbm, out_hbm,
               xv, idx_v, vals_v, offs_v, bias_v, out_v,
               sem_x, *seg_sems):
    wid = lax.axis_index("s") * NUM_CORES + lax.axis_index("c")
    base = wid * B_PER_W

    x_dma = pltpu.async_copy(xt_hbm.at[:, pl.ds(base, B_PER_W)], xv, sem_x)
    pltpu.sync_copy(offs_hbm, offs_v.at[pl.ds(8, NUM_FIELDS)])
    pltpu.sync_copy(bias_hbm, bias_v.at[pl.ds(8, 1)])

    off_vecs = [
        plsc.load_gather(offs_v, [lax.full((LANES,), 8 + f, jnp.int32)])
        for f in range(NUM_FIELDS)
    ]
    x_dma.wait()

    def build_chunk(c, carry):
        sl = pl.ds(c * LANES, LANES)
        for f in range(NUM_FIELDS):
            idx_v[pl.ds(c * CHUNK_IDX + f * LANES, LANES)] = (
                xv[f, sl] + off_vecs[f]
            )
        return carry

    def fire_segment(s, sem):
        lax.fori_loop(s * SEG_CHUNKS, (s + 1) * SEG_CHUNKS, build_chunk, 0)
        return pltpu.async_copy(
            table_hbm.at[0].at[idx_v.at[pl.ds(s * SEG_IDX, SEG_IDX)]],
            vals_v.at[pl.ds(s * SEG_IDX, SEG_IDX)],
            sem,
        )

    dmas = [fire_segment(s, seg_sems[s]) for s in range(NUM_SEGS)]

    bias_vec = plsc.load_gather(bias_v, [lax.full((LANES,), 8, jnp.int32)])

    def reduce_chunk(c, carry):
        acc = bias_vec
        for f in range(NUM_FIELDS):
            acc = acc + vals_v[pl.ds(c * CHUNK_IDX + f * LANES, LANES)]
        out_v[pl.ds(c * LANES, LANES)] = acc
        return carry

    for s in range(NUM_SEGS):
        dmas[s].wait()
        lax.fori_loop(s * SEG_CHUNKS, (s + 1) * SEG_CHUNKS, reduce_chunk, 0)

    pltpu.sync_copy(out_v, out_hbm.at[pl.ds(base, B_PER_W)])


def kernel(x, offsets, fc_weight, bias):
    xt = x.astype(jnp.int32).T
    offs = offsets.astype(jnp.int32)
    table = fc_weight.T
    out = _fl_kernel(xt, offs, table, bias)
    return out.reshape(BATCH, 1)


# speedup vs baseline: 1.0725x; 1.0725x over previous
import functools

import jax
import jax.numpy as jnp
from jax import lax
from jax.experimental import pallas as pl
from jax.experimental.pallas import tpu as pltpu
from jax.experimental.pallas import tpu_sc as plsc

BATCH = 16384
NUM_FIELDS = 26
TOTAL_ROWS = 100000 * 26

NUM_CORES = 2
NUM_SUBCORES = 16
LANES = 16
NUM_WORKERS = NUM_CORES * NUM_SUBCORES

B_PER_W = BATCH // NUM_WORKERS
IDX_PER_W = B_PER_W * NUM_FIELDS
B_CHUNKS = B_PER_W // LANES
CHUNK_IDX = NUM_FIELDS * LANES
NUM_SEGS = 4
SEG_CHUNKS = B_CHUNKS // NUM_SEGS
SEG_IDX = SEG_CHUNKS * CHUNK_IDX


@functools.partial(
    pl.kernel,
    out_type=jax.ShapeDtypeStruct((BATCH,), jnp.float32),
    mesh=plsc.VectorSubcoreMesh(core_axis_name="c", subcore_axis_name="s"),
    compiler_params=pltpu.CompilerParams(needs_layout_passes=False),
    scratch_types=[
        pltpu.VMEM((NUM_FIELDS, B_PER_W), jnp.int32),
        pltpu.VMEM((IDX_PER_W,), jnp.int32),
        pltpu.VMEM((IDX_PER_W,), jnp.float32),
        pltpu.VMEM((8 + NUM_FIELDS,), jnp.int32),
        pltpu.VMEM((16,), jnp.float32),
        pltpu.VMEM((B_PER_W,), jnp.float32),
        pltpu.SemaphoreType.DMA,
        pltpu.SemaphoreType.DMA,
        pltpu.SemaphoreType.DMA,
        pltpu.SemaphoreType.DMA,
        pltpu.SemaphoreType.DMA,
    ],
)
def _fl_kernel(xt_hbm, offs_hbm, table_hbm, bias_hbm, out_hbm,
               xv, idx_v, vals_v, offs_v, bias_v, out_v,
               sem_x, sem0, sem1, sem2, sem3):
    wid = lax.axis_index("s") * NUM_CORES + lax.axis_index("c")
    base = wid * B_PER_W
    seg_sems = (sem0, sem1, sem2, sem3)

    x_dma = pltpu.async_copy(xt_hbm.at[:, pl.ds(base, B_PER_W)], xv, sem_x)
    pltpu.sync_copy(offs_hbm, offs_v.at[pl.ds(8, NUM_FIELDS)])
    pltpu.sync_copy(bias_hbm, bias_v.at[pl.ds(8, 1)])

    off_vecs = [
        plsc.load_gather(offs_v, [lax.full((LANES,), 8 + f, jnp.int32)])
        for f in range(NUM_FIELDS)
    ]
    x_dma.wait()

    def build_chunk(c, carry):
        sl = pl.ds(c * LANES, LANES)
        for f in range(NUM_FIELDS):
            idx_v[pl.ds(c * CHUNK_IDX + f * LANES, LANES)] = (
                xv[f, sl] + off_vecs[f]
            )
        return carry

    def fire_segment(s, sem):
        lax.fori_loop(s * SEG_CHUNKS, (s + 1) * SEG_CHUNKS, build_chunk, 0)
        return pltpu.async_copy(
            table_hbm.at[0].at[idx_v.at[pl.ds(s * SEG_IDX, SEG_IDX)]],
            vals_v.at[pl.ds(s * SEG_IDX, SEG_IDX)],
            sem,
        )

    dmas = [fire_segment(s, seg_sems[s]) for s in range(NUM_SEGS)]

    bias_vec = plsc.load_gather(bias_v, [lax.full((LANES,), 8, jnp.int32)])

    def reduce_chunk(c, carry):
        acc = bias_vec
        for f in range(NUM_FIELDS):
            acc = acc + vals_v[pl.ds(c * CHUNK_IDX + f * LANES, LANES)]
        out_v[pl.ds(c * LANES, LANES)] = acc
        return carry

    for s in range(NUM_SEGS):
        dmas[s].wait()
        lax.fori_loop(s * SEG_CHUNKS, (s + 1) * SEG_CHUNKS, reduce_chunk, 0)

    pltpu.sync_copy(out_v, out_hbm.at[pl.ds(base, B_PER_W)])


def kernel(x, offsets, fc_weight, bias):
    xt = x.astype(jnp.int32).T
    offs = offsets.astype(jnp.int32)
    table = fc_weight.T
    out = _fl_kernel(xt, offs, table, bias)
    return out.reshape(BATCH, 1)
